# baseline (device time: 88884 ns/iter reference)
import jax
import jax.numpy as jnp
from jax import lax
from jax.experimental import pallas as pl
from jax.experimental.pallas import tpu as pltpu

_BLK = 512
_M = 64
_D = 2048
_H = 4096

_HALVES = (
    (_D, _H, "y", True),
    (_H, _D, "x", False),
    (_D, _H, "y", True),
    (_H, _D, "x", False),
    (_D, _H, "y", True),
    (_H, _D, "x", False),
)
_NCHUNKS = tuple(n // _BLK for (_, n, _, _) in _HALVES)
_MAXC = max(_NCHUNKS)
_TILES = tuple((i, j) for i in range(6) for j in range(_NCHUNKS[i]))


def kernel(x, Win0, Wout0, Win1, Wout1, Win2, Wout2):
    def body(x_ref, w0, w1, w2, w3, w4, w5, out_ref,
             wbuf, wsems, acc, recv, send_sems, recv_sems, h_ref, x2_ref):
        w_refs = (w0, w1, w2, w3, w4, w5)
        my_x = lax.axis_index("x")
        my_y = lax.axis_index("y")
        peers = {"y": (my_x, 1 - my_y), "x": (1 - my_x, my_y)}

        barrier_sem = pltpu.get_barrier_semaphore()
        for ax in ("y", "x"):
            pl.semaphore_signal(
                barrier_sem, inc=1, device_id=peers[ax],
                device_id_type=pl.DeviceIdType.MESH,
            )
        pl.semaphore_wait(barrier_sem, 2)

        def wcopy(t):
            i, j = _TILES[t]
            k = _HALVES[i][0]
            slot = t % 2
            return pltpu.make_async_copy(
                w_refs[i].at[:, pl.ds(j * _BLK, _BLK)],
                wbuf.at[slot, pl.ds(0, k), :],
                wsems.at[slot],
            )

        copies = {0: wcopy(0)}
        copies[0].start()

        act_refs = {0: x_ref, 1: h_ref, 2: x2_ref, 3: h_ref,
                    4: x2_ref, 5: h_ref}
        dst_refs = {0: h_ref, 1: x2_ref, 2: h_ref, 3: x2_ref,
                    4: h_ref, 5: out_ref}

        t = 0
        for i, (k, n, ax, relu) in enumerate(_HALVES):
            peer = peers[ax]
            rdmas = []
            for j in range(_NCHUNKS[i]):
                if t + 1 < len(_TILES):
                    copies[t + 1] = wcopy(t + 1)
                    copies[t + 1].start()
                copies[t].wait()
                acc[j, :, :] = jnp.dot(
                    act_refs[i][:, 0:k], wbuf[t % 2, 0:k, :],
                    preferred_element_type=jnp.float32,
                )
                rdma = pltpu.make_async_remote_copy(
                    src_ref=acc.at[j],
                    dst_ref=recv.at[i, j],
                    send_sem=send_sems.at[i, j],
                    recv_sem=recv_sems.at[i, j],
                    device_id=peer,
                    device_id_type=pl.DeviceIdType.MESH,
                )
                rdma.start()
                rdmas.append(rdma)
                t += 1
            for j, r in enumerate(rdmas):
                r.wait()
                s = acc[j, :, :] + recv[i, j, :, :]
                if relu:
                    s = jnp.maximum(s, 0.0)
                dst_refs[i][:, pl.ds(j * _BLK, _BLK)] = s

    out_shape = jax.ShapeDtypeStruct((_M, _D), jnp.float32)
    return pl.pallas_call(
        body,
        out_shape=out_shape,
        in_specs=[pl.BlockSpec(memory_space=pltpu.VMEM)]
        + [pl.BlockSpec(memory_space=pl.ANY)] * 6,
        out_specs=pl.BlockSpec(memory_space=pltpu.VMEM),
        scratch_shapes=[
            pltpu.VMEM((2, _H, _BLK), jnp.float32),
            pltpu.SemaphoreType.DMA((2,)),
            pltpu.VMEM((_MAXC, _M, _BLK), jnp.float32),
            pltpu.VMEM((6, _MAXC, _M, _BLK), jnp.float32),
            pltpu.SemaphoreType.DMA((6, _MAXC)),
            pltpu.SemaphoreType.DMA((6, _MAXC)),
            pltpu.VMEM((_M, _H), jnp.float32),
            pltpu.VMEM((_M, _D), jnp.float32),
        ],
        compiler_params=pltpu.CompilerParams(collective_id=0),
    )(x, Win0, Wout0, Win1, Wout1, Win2, Wout2)


# device time: 79605 ns/iter; 1.1166x vs baseline; 1.1166x over previous
import jax
import jax.numpy as jnp
from jax import lax
from jax.experimental import pallas as pl
from jax.experimental.pallas import tpu as pltpu

_BLK = 512
_NSLOTS = 3
_M = 64
_D = 2048
_H = 4096

_HALVES = (
    (_D, _H, "y", True),
    (_H, _D, "x", False),
    (_D, _H, "y", True),
    (_H, _D, "x", False),
    (_D, _H, "y", True),
    (_H, _D, "x", False),
)
_NCHUNKS = tuple(n // _BLK for (_, n, _, _) in _HALVES)
_MAXC = max(_NCHUNKS)
_TILES = tuple((i, j) for i in range(6) for j in range(_NCHUNKS[i]))


def kernel(x, Win0, Wout0, Win1, Wout1, Win2, Wout2):
    def body(x_ref, w0, w1, w2, w3, w4, w5, out_ref,
             wbuf, wsems, acc, recv, send_sems, recv_sems, h_ref, x2_ref):
        w_refs = (w0, w1, w2, w3, w4, w5)
        my_x = lax.axis_index("x")
        my_y = lax.axis_index("y")
        peers = {"y": (my_x, 1 - my_y), "x": (1 - my_x, my_y)}

        barrier_sem = pltpu.get_barrier_semaphore()
        for ax in ("y", "x"):
            pl.semaphore_signal(
                barrier_sem, inc=1, device_id=peers[ax],
                device_id_type=pl.DeviceIdType.MESH,
            )
        pl.semaphore_wait(barrier_sem, 2)

        def wcopy(t):
            i, j = _TILES[t]
            k = _HALVES[i][0]
            slot = t % _NSLOTS
            return pltpu.make_async_copy(
                w_refs[i].at[:, pl.ds(j * _BLK, _BLK)],
                wbuf.at[slot, pl.ds(0, k), :],
                wsems.at[slot],
            )

        copies = {}
        for s in range(_NSLOTS - 1):
            copies[s] = wcopy(s)
            copies[s].start()

        act_refs = {0: x_ref, 1: h_ref, 2: x2_ref, 3: h_ref,
                    4: x2_ref, 5: h_ref}
        dst_refs = {0: h_ref, 1: x2_ref, 2: h_ref, 3: x2_ref,
                    4: h_ref, 5: out_ref}

        t = 0
        for i, (k, n, ax, relu) in enumerate(_HALVES):
            peer = peers[ax]
            rdmas = []
            for j in range(_NCHUNKS[i]):
                if t + _NSLOTS - 1 < len(_TILES):
                    copies[t + _NSLOTS - 1] = wcopy(t + _NSLOTS - 1)
                    copies[t + _NSLOTS - 1].start()
                copies[t].wait()
                acc[j, :, :] = jnp.dot(
                    act_refs[i][:, 0:k], wbuf[t % _NSLOTS, 0:k, :],
                    preferred_element_type=jnp.float32,
                )
                rdma = pltpu.make_async_remote_copy(
                    src_ref=acc.at[j],
                    dst_ref=recv.at[i, j],
                    send_sem=send_sems.at[i, j],
                    recv_sem=recv_sems.at[i, j],
                    device_id=peer,
                    device_id_type=pl.DeviceIdType.MESH,
                )
                rdma.start()
                rdmas.append(rdma)
                t += 1
            for j, r in enumerate(rdmas):
                r.wait()
                s = acc[j, :, :] + recv[i, j, :, :]
                if relu:
                    s = jnp.maximum(s, 0.0)
                dst_refs[i][:, pl.ds(j * _BLK, _BLK)] = s

    out_shape = jax.ShapeDtypeStruct((_M, _D), jnp.float32)
    return pl.pallas_call(
        body,
        out_shape=out_shape,
        in_specs=[pl.BlockSpec(memory_space=pltpu.VMEM)]
        + [pl.BlockSpec(memory_space=pl.ANY)] * 6,
        out_specs=pl.BlockSpec(memory_space=pltpu.VMEM),
        scratch_shapes=[
            pltpu.VMEM((_NSLOTS, _H, _BLK), jnp.float32),
            pltpu.SemaphoreType.DMA((_NSLOTS,)),
            pltpu.VMEM((_MAXC, _M, _BLK), jnp.float32),
            pltpu.VMEM((6, _MAXC, _M, _BLK), jnp.float32),
            pltpu.SemaphoreType.DMA((6, _MAXC)),
            pltpu.SemaphoreType.DMA((6, _MAXC)),
            pltpu.VMEM((_M, _H), jnp.float32),
            pltpu.VMEM((_M, _D), jnp.float32),
        ],
        compiler_params=pltpu.CompilerParams(
            collective_id=0, vmem_limit_bytes=36 * 1024 * 1024),
    )(x, Win0, Wout0, Win1, Wout1, Win2, Wout2)
